# baseline (speedup 1.0000x reference)
import functools

import jax
import jax.numpy as jnp
from jax import lax
from jax.experimental import pallas as pl
from jax.experimental.pallas import tpu as pltpu
from jax.experimental.pallas import tpu_sc as plsc

_NC = 2
_NS = 16
_NW = _NC * _NS
_LANES = 16


def _tc_table_to_rowmajor(tab_t, V, D, vchunk=8192):
    grid = (V + vchunk - 1) // vchunk

    def body(i_ref, o_ref):
        eye = jnp.eye(D, dtype=jnp.float32)
        t = jax.lax.dot_general(
            i_ref[...], eye, (((0,), (0,)), ((), ())),
            preferred_element_type=jnp.float32)
        t3 = t.reshape(vchunk // 2, 2, D)
        o_ref[...] = jnp.concatenate([t3[:, 0, :], t3[:, 1, :]], axis=1)

    return pl.pallas_call(
        body,
        grid=(grid,),
        in_specs=[pl.BlockSpec((D, vchunk), lambda i: (0, i))],
        out_specs=pl.BlockSpec((vchunk // 2, 2 * D), lambda i: (i, 0)),
        out_shape=jax.ShapeDtypeStruct((V // 2, 2 * D), jnp.float32),
    )(tab_t)


def _make_sc_kernel(Bsz, Lsz, V, D, seq_per_w, cs, g_sub):
    rc = cs * Lsz
    nsub = Lsz // g_sub
    nch = seq_per_w // cs

    mesh = plsc.VectorSubcoreMesh(core_axis_name="c", subcore_axis_name="s")

    def body(idx_hbm, tab_hbm, pos_hbm, out_hbm,
             idx_v, pos_v, buf_a, buf_b, gsem_a, gsem_b, osem_a, osem_b):
        c = lax.axis_index("c")
        s = lax.axis_index("s")
        wid = s * _NC + c
        seq0 = wid * seq_per_w
        pltpu.sync_copy(idx_hbm.at[pl.ds(seq0, seq_per_w)], idx_v)
        pltpu.sync_copy(pos_hbm, pos_v)

        def fire_gathers(buf, sem, ch):
            hs = []
            for sq in range(cs):
                for k in range(nsub):
                    hs.append(pltpu.async_copy(
                        tab_hbm.at[idx_v.at[ch * cs + sq, pl.ds(k * g_sub, g_sub)]],
                        buf.at[pl.ds(sq * Lsz + k * g_sub, g_sub)],
                        sem,
                    ))
            return hs

        def add_pos(buf):
            for sq in range(cs):
                base = sq * Lsz

                @plsc.parallel_loop(0, Lsz, unroll=4)
                def _(r):
                    for cg in range(D // _LANES):
                        sl = pl.ds(cg * _LANES, _LANES)
                        buf[base + r, sl] = buf[base + r, sl] + pos_v[r, sl]

        def store_out(buf, sem, ch):
            return [
                pltpu.async_copy(
                    buf.at[pl.ds(sq * Lsz, Lsz)],
                    out_hbm.at[seq0 + ch * cs + sq],
                    sem,
                )
                for sq in range(cs)
            ]

        def pair_body(g2, carry):
            ch_a = g2 * 2
            ch_b = ch_a + 1
            hs_a = fire_gathers(buf_a, gsem_a, ch_a)
            hs_b = fire_gathers(buf_b, gsem_b, ch_b)
            for h in hs_a:
                h.wait()
            add_pos(buf_a)
            out_a = store_out(buf_a, osem_a, ch_a)
            for h in hs_b:
                h.wait()
            add_pos(buf_b)
            out_b = store_out(buf_b, osem_b, ch_b)
            for h in out_a + out_b:
                h.wait()
            return carry

        lax.fori_loop(0, nch // 2, pair_body, None)

    return pl.kernel(
        body,
        out_type=jax.ShapeDtypeStruct((Bsz, Lsz, D), jnp.float32),
        mesh=mesh,
        compiler_params=pltpu.CompilerParams(
            use_tc_tiling_on_sc=False, needs_layout_passes=False,
            skip_device_barrier=True),
        scratch_types=[
            pltpu.VMEM((seq_per_w, Lsz), jnp.int32),
            pltpu.VMEM((Lsz, D), jnp.float32),
            pltpu.VMEM((rc, D), jnp.float32),
            pltpu.VMEM((rc, D), jnp.float32),
            pltpu.SemaphoreType.DMA,
            pltpu.SemaphoreType.DMA,
            pltpu.SemaphoreType.DMA,
            pltpu.SemaphoreType.DMA,
        ],
    )


def _tc_out_to_tiled(out_p, Bsz, Lsz, D):
    bt = Bsz // 128
    dh = D // 8
    lc2 = Lsz // 2
    rows = 128 * lc2

    def body(i_ref, o_ref):
        blk = i_ref[...]
        blk = blk.reshape(128, lc2, 2 * D)
        t = jnp.transpose(blk, (1, 2, 0))
        t = t.reshape(lc2, 2, D, 128)
        t = t.reshape(Lsz, D, 128)
        t = t.reshape(Lsz, dh, 8, 128)
        o_ref[...] = t.reshape(Lsz, dh, 1, 8, 128)

    return pl.pallas_call(
        body,
        grid=(bt,),
        in_specs=[pl.BlockSpec((rows, 2 * D), lambda b: (b, 0))],
        out_specs=pl.BlockSpec(
            (Lsz, dh, 1, 8, 128), lambda b: (0, 0, b, 0, 0)),
        out_shape=jax.ShapeDtypeStruct((Lsz, dh, bt, 8, 128), jnp.float32),
        compiler_params=pltpu.CompilerParams(
            vmem_limit_bytes=56 * 1024 * 1024),
    )(out_p)


def kernel(x, word_table, pos_table):
    Bsz, Lsz = x.shape
    V, D = word_table.shape
    seq_per_w = Bsz // _NW
    cs = 2
    g_sub = 40

    pos_rows = pos_table[1 : Lsz + 1]

    tab_p = _tc_table_to_rowmajor(jnp.transpose(word_table), V, D)
    tab_rm = tab_p.reshape(V, D)

    sc = _make_sc_kernel(Bsz, Lsz, V, D, seq_per_w, cs, g_sub)
    out_rm = sc(x, tab_rm, pos_rows)

    out5 = _tc_out_to_tiled(
        out_rm.reshape(Bsz * Lsz // 2, 2 * D), Bsz, Lsz, D)
    return jnp.transpose(out5, (2, 4, 0, 1, 3)).reshape(Bsz, Lsz, D)


# speedup vs baseline: 1.0465x; 1.0465x over previous
import functools

import jax
import jax.numpy as jnp
from jax import lax
from jax.experimental import pallas as pl
from jax.experimental.pallas import tpu as pltpu
from jax.experimental.pallas import tpu_sc as plsc

_NC = 2
_NS = 16
_NW = _NC * _NS
_LANES = 16


def _tc_table_to_rowmajor(tab_t, V, D, vchunk=16384):
    grid = (V + vchunk - 1) // vchunk

    def body(i_ref, o_ref):
        t3 = i_ref[...].T.reshape(vchunk // 2, 2, D)
        o_ref[...] = jnp.concatenate([t3[:, 0, :], t3[:, 1, :]], axis=1)

    return pl.pallas_call(
        body,
        grid=(grid,),
        in_specs=[pl.BlockSpec((D, vchunk), lambda i: (0, i))],
        out_specs=pl.BlockSpec((vchunk // 2, 2 * D), lambda i: (i, 0)),
        out_shape=jax.ShapeDtypeStruct((V // 2, 2 * D), jnp.float32),
    )(tab_t)


def _make_sc_kernel(Bsz, Lsz, V, D, seq_per_w, cs, g_sub):
    rc = cs * Lsz
    nsub = Lsz // g_sub
    nch = seq_per_w // cs

    mesh = plsc.VectorSubcoreMesh(core_axis_name="c", subcore_axis_name="s")

    def body(idx_hbm, tab_hbm, pos_hbm, out_hbm,
             idx_v, pos_v, buf_a, buf_b, gsem_a, gsem_b, osem_a, osem_b):
        c = lax.axis_index("c")
        s = lax.axis_index("s")
        wid = s * _NC + c
        seq0 = wid * seq_per_w
        pltpu.sync_copy(idx_hbm.at[pl.ds(seq0, seq_per_w)], idx_v)
        pltpu.sync_copy(pos_hbm, pos_v)

        def fire_gathers(buf, sem, ch):
            hs = []
            for sq in range(cs):
                for k in range(nsub):
                    hs.append(pltpu.async_copy(
                        tab_hbm.at[idx_v.at[ch * cs + sq, pl.ds(k * g_sub, g_sub)]],
                        buf.at[pl.ds(sq * Lsz + k * g_sub, g_sub)],
                        sem,
                    ))
            return hs

        def add_pos(buf):
            for sq in range(cs):
                base = sq * Lsz

                @plsc.parallel_loop(0, Lsz, unroll=4)
                def _(r):
                    for cg in range(D // _LANES):
                        sl = pl.ds(cg * _LANES, _LANES)
                        buf[base + r, sl] = buf[base + r, sl] + pos_v[r, sl]

        def store_out(buf, sem, ch):
            return [
                pltpu.async_copy(
                    buf.at[pl.ds(sq * Lsz, Lsz)],
                    out_hbm.at[seq0 + ch * cs + sq],
                    sem,
                )
                for sq in range(cs)
            ]

        def pair_body(g2, carry):
            ch_a = g2 * 2
            ch_b = ch_a + 1
            hs_a = fire_gathers(buf_a, gsem_a, ch_a)
            hs_b = fire_gathers(buf_b, gsem_b, ch_b)
            for h in hs_a:
                h.wait()
            add_pos(buf_a)
            out_a = store_out(buf_a, osem_a, ch_a)
            for h in hs_b:
                h.wait()
            add_pos(buf_b)
            out_b = store_out(buf_b, osem_b, ch_b)
            for h in out_a + out_b:
                h.wait()
            return carry

        lax.fori_loop(0, nch // 2, pair_body, None)

    return pl.kernel(
        body,
        out_type=jax.ShapeDtypeStruct((Bsz, Lsz, D), jnp.float32),
        mesh=mesh,
        compiler_params=pltpu.CompilerParams(
            use_tc_tiling_on_sc=False, needs_layout_passes=False,
            skip_device_barrier=True),
        scratch_types=[
            pltpu.VMEM((seq_per_w, Lsz), jnp.int32),
            pltpu.VMEM((Lsz, D), jnp.float32),
            pltpu.VMEM((rc, D), jnp.float32),
            pltpu.VMEM((rc, D), jnp.float32),
            pltpu.SemaphoreType.DMA,
            pltpu.SemaphoreType.DMA,
            pltpu.SemaphoreType.DMA,
            pltpu.SemaphoreType.DMA,
        ],
    )


def _tc_out_to_tiled(out_p, Bsz, Lsz, D):
    bt = Bsz // 128
    dh = D // 8
    lc2 = Lsz // 2
    rows = 128 * lc2

    def body(i_ref, o_ref):
        blk = i_ref[...]
        blk = blk.reshape(128, lc2, 2 * D)
        t = jnp.transpose(blk, (1, 2, 0))
        t = t.reshape(lc2, 2, D, 128)
        t = t.reshape(Lsz, D, 128)
        t = t.reshape(Lsz, dh, 8, 128)
        o_ref[...] = t.reshape(Lsz, dh, 1, 8, 128)

    return pl.pallas_call(
        body,
        grid=(bt,),
        in_specs=[pl.BlockSpec((rows, 2 * D), lambda b: (b, 0))],
        out_specs=pl.BlockSpec(
            (Lsz, dh, 1, 8, 128), lambda b: (0, 0, b, 0, 0)),
        out_shape=jax.ShapeDtypeStruct((Lsz, dh, bt, 8, 128), jnp.float32),
        compiler_params=pltpu.CompilerParams(
            vmem_limit_bytes=56 * 1024 * 1024),
    )(out_p)


def kernel(x, word_table, pos_table):
    Bsz, Lsz = x.shape
    V, D = word_table.shape
    seq_per_w = Bsz // _NW
    cs = 2
    g_sub = 40

    pos_rows = pos_table[1 : Lsz + 1]

    tab_p = _tc_table_to_rowmajor(jnp.transpose(word_table), V, D)
    tab_rm = tab_p.reshape(V, D)

    sc = _make_sc_kernel(Bsz, Lsz, V, D, seq_per_w, cs, g_sub)
    out_rm = sc(x, tab_rm, pos_rows)

    out5 = _tc_out_to_tiled(
        out_rm.reshape(Bsz * Lsz // 2, 2 * D), Bsz, Lsz, D)
    return jnp.transpose(out5, (2, 4, 0, 1, 3)).reshape(Bsz, Lsz, D)
